# baseline (speedup 1.0000x reference)
import functools

import jax
import jax.numpy as jnp
from jax import lax
from jax.experimental import pallas as pl
from jax.experimental.pallas import tpu as pltpu
from jax.experimental.pallas import tpu_sc as plsc

N = 10000
D = 128
E = 320000
NC, NS = 2, 16
ROWS_PER_TILE = 640
ROWS_PAD = NS * ROWS_PER_TILE
DUMMY_ROW = ROWS_PAD - 1
CHUNK = 128
CH_PER_TILE = 80
HALF = CH_PER_TILE // 2
E_PAD = NC * NS * CH_PER_TILE * CHUNK


def _mm_body(x_ref, w_ref, o_ref):
    o_ref[...] = jnp.dot(x_ref[...], w_ref[...],
                         preferred_element_type=jnp.float32)


def _matmul(x, w):
    m = x.shape[0]
    bm = 1280
    return pl.pallas_call(
        _mm_body,
        grid=(m // bm,),
        in_specs=[pl.BlockSpec((bm, D), lambda i: (i, 0)),
                  pl.BlockSpec((D, D), lambda i: (0, 0))],
        out_specs=pl.BlockSpec((bm, D), lambda i: (i, 0)),
        out_shape=jax.ShapeDtypeStruct((m, D), jnp.float32),
    )(x, w)


def _fused_body(p0_ref, p1_ref, b_ref, w_ref, o_ref):
    h = jnp.maximum(p0_ref[...] + p1_ref[...] + b_ref[0:1, :], 0.0)
    o_ref[...] = jnp.dot(h, w_ref[...], preferred_element_type=jnp.float32)


def _fused_relu_mm(p0, p1, b, w):
    m = p0.shape[0]
    bm = 1280
    return pl.pallas_call(
        _fused_body,
        grid=(m // bm,),
        in_specs=[pl.BlockSpec((bm, D), lambda i: (i, 0)),
                  pl.BlockSpec((bm, D), lambda i: (i, 0)),
                  pl.BlockSpec((8, D), lambda i: (0, 0)),
                  pl.BlockSpec((D, D), lambda i: (0, 0))],
        out_specs=pl.BlockSpec((bm, D), lambda i: (i, 0)),
        out_shape=jax.ShapeDtypeStruct((m, D), jnp.float32),
    )(p0, p1, b, w)


def _add_body(p0_ref, p1_ref, b_ref, o_ref):
    o_ref[...] = p0_ref[...] + p1_ref[...] + b_ref[0:1, :]


def _final_add(p0, p1, b):
    bm = 1000
    return pl.pallas_call(
        _add_body,
        grid=(N // bm,),
        in_specs=[pl.BlockSpec((bm, D), lambda i: (i, 0)),
                  pl.BlockSpec((bm, D), lambda i: (i, 0)),
                  pl.BlockSpec((8, D), lambda i: (0, 0))],
        out_specs=pl.BlockSpec((bm, D), lambda i: (i, 0)),
        out_shape=jax.ShapeDtypeStruct((N, D), jnp.float32),
    )(p0, p1, b)


def _make_edge_scatter():
    mesh = plsc.VectorSubcoreMesh(core_axis_name="c", subcore_axis_name="s")

    @functools.partial(
        pl.kernel,
        out_type=jax.ShapeDtypeStruct((NC, ROWS_PAD, D), jnp.float32),
        mesh=mesh,
        scratch_types=[
            pltpu.VMEM((HALF, CHUNK), jnp.int32),
            pltpu.VMEM((HALF, CHUNK), jnp.int32),
            pltpu.VMEM((CHUNK, D), jnp.float32),
            pltpu.VMEM((CHUNK, D), jnp.float32),
            pltpu.VMEM_SHARED((ROWS_PAD, D), jnp.float32),
            pltpu.SemaphoreType.DMA,
            pltpu.SemaphoreType.DMA,
        ],
    )
    def edge_scatter(y_hbm, src_hbm, dst_hbm, zeros_hbm, out_hbm,
                     src_v, dst_v, rows_a, rows_b, acc_sh, gsem, ssem):
        c = lax.axis_index("c")
        s = lax.axis_index("s")
        stripe = pl.ds(s * ROWS_PER_TILE, ROWS_PER_TILE)
        pltpu.sync_copy(zeros_hbm, acc_sh.at[stripe])

        def gather(j, buf):
            pltpu.async_copy(y_hbm.at[src_v.at[j]], buf, gsem)

        def gather_wait(j, buf):
            pltpu.make_async_copy(y_hbm.at[src_v.at[j]], buf, gsem).wait()

        def scatter(j, buf):
            pltpu.async_copy(buf, acc_sh.at[dst_v.at[j]], ssem, add=True)

        def scatter_wait(j, buf):
            pltpu.make_async_copy(buf, acc_sh.at[dst_v.at[j]], ssem).wait()

        for p in range(2):
            pltpu.sync_copy(src_hbm.at[c, s, pl.ds(p * HALF, HALF)], src_v)
            pltpu.sync_copy(dst_hbm.at[c, s, pl.ds(p * HALF, HALF)], dst_v)
            if p == 0:
                plsc.subcore_barrier()

            gather(0, rows_a)
            gather_wait(0, rows_a)
            scatter(0, rows_a)
            gather(1, rows_b)
            gather_wait(1, rows_b)
            scatter(1, rows_b)
            scatter_wait(0, rows_a)
            gather(2, rows_a)

            @pl.loop(2, HALF - 2, step=2)
            def _(j):
                gather_wait(j, rows_a)
                scatter(j, rows_a)
                scatter_wait(j - 1, rows_b)
                gather(j + 1, rows_b)
                gather_wait(j + 1, rows_b)
                scatter(j + 1, rows_b)
                scatter_wait(j, rows_a)
                gather(j + 2, rows_a)

            gather_wait(HALF - 2, rows_a)
            scatter(HALF - 2, rows_a)
            scatter_wait(HALF - 3, rows_b)
            gather(HALF - 1, rows_b)
            gather_wait(HALF - 1, rows_b)
            scatter(HALF - 1, rows_b)
            scatter_wait(HALF - 2, rows_a)
            scatter_wait(HALF - 1, rows_b)

        plsc.subcore_barrier()
        pltpu.sync_copy(acc_sh.at[stripe], out_hbm.at[c, stripe])

    return edge_scatter


_edge_scatter = _make_edge_scatter()


def kernel(x, edge_index, W1, b1, W2, b2):
    src = edge_index[0].astype(jnp.int32)
    dst = edge_index[1].astype(jnp.int32)
    padn = E_PAD - E
    pad_src = jnp.arange(padn, dtype=jnp.int32) % N
    src_p = jnp.concatenate([src, pad_src]).reshape(NC, NS, CH_PER_TILE, CHUNK)
    dummy = N + (jnp.arange(padn, dtype=jnp.int32) % (ROWS_PAD - N))
    dst_p = jnp.concatenate([dst, dummy]).reshape(NC, NS, CH_PER_TILE, CHUNK)
    zeros_tile = jnp.zeros((ROWS_PER_TILE, D), jnp.float32)
    b1_t = jnp.broadcast_to(b1.reshape(1, D), (8, D))
    b2_t = jnp.broadcast_to(b2.reshape(1, D), (8, D))

    x_pad = jnp.zeros((ROWS_PAD, D), jnp.float32).at[:N].set(x)
    y1 = _matmul(x_pad, W1)
    p1 = _edge_scatter(y1, src_p, dst_p, zeros_tile)
    y2 = _fused_relu_mm(p1[0], p1[1], b1_t, W2)
    p2 = _edge_scatter(y2, src_p, dst_p, zeros_tile)
    return _final_add(p2[0], p2[1], b2_t)


# speedup vs baseline: 1.0541x; 1.0541x over previous
import functools

import jax
import jax.numpy as jnp
from jax import lax
from jax.experimental import pallas as pl
from jax.experimental.pallas import tpu as pltpu
from jax.experimental.pallas import tpu_sc as plsc

N = 10000
D = 128
E = 320000
NC, NS = 2, 16
ROWS_PER_TILE = 640
ROWS_PAD = NS * ROWS_PER_TILE
DUMMY_ROW = ROWS_PAD - 1
CHUNK = 128
CH_PER_TILE = 80
HALF = CH_PER_TILE // 2
E_PAD = NC * NS * CH_PER_TILE * CHUNK


def _mm_body(x_ref, w_ref, o_ref):
    o_ref[...] = jnp.dot(x_ref[...], w_ref[...],
                         preferred_element_type=jnp.float32)


def _matmul(x, w):
    m = x.shape[0]
    bm = 2000
    return pl.pallas_call(
        _mm_body,
        grid=(m // bm,),
        in_specs=[pl.BlockSpec((bm, D), lambda i: (i, 0)),
                  pl.BlockSpec((D, D), lambda i: (0, 0))],
        out_specs=pl.BlockSpec((bm, D), lambda i: (i, 0)),
        out_shape=jax.ShapeDtypeStruct((m, D), jnp.float32),
    )(x, w)


def _fused_body(p0_ref, p1_ref, b_ref, w_ref, o_ref):
    h = jnp.maximum(p0_ref[0] + p1_ref[0] + b_ref[0:1, :], 0.0)
    o_ref[...] = jnp.dot(h, w_ref[...], preferred_element_type=jnp.float32)


def _fused_relu_mm(p, b, w):
    m = p.shape[1]
    bm = 1280
    return pl.pallas_call(
        _fused_body,
        grid=(m // bm,),
        in_specs=[pl.BlockSpec((1, bm, D), lambda i: (0, i, 0)),
                  pl.BlockSpec((1, bm, D), lambda i: (1, i, 0)),
                  pl.BlockSpec((8, D), lambda i: (0, 0)),
                  pl.BlockSpec((D, D), lambda i: (0, 0))],
        out_specs=pl.BlockSpec((bm, D), lambda i: (i, 0)),
        out_shape=jax.ShapeDtypeStruct((m, D), jnp.float32),
    )(p, p, b, w)


def _add_body(p0_ref, p1_ref, b_ref, o_ref):
    o_ref[...] = p0_ref[0] + p1_ref[0] + b_ref[0:1, :]


def _final_add(p, b):
    bm = 1000
    return pl.pallas_call(
        _add_body,
        grid=(N // bm,),
        in_specs=[pl.BlockSpec((1, bm, D), lambda i: (0, i, 0)),
                  pl.BlockSpec((1, bm, D), lambda i: (1, i, 0)),
                  pl.BlockSpec((8, D), lambda i: (0, 0))],
        out_specs=pl.BlockSpec((bm, D), lambda i: (i, 0)),
        out_shape=jax.ShapeDtypeStruct((N, D), jnp.float32),
    )(p, p, b)


def _make_edge_scatter():
    mesh = plsc.VectorSubcoreMesh(core_axis_name="c", subcore_axis_name="s")

    @functools.partial(
        pl.kernel,
        out_type=jax.ShapeDtypeStruct((NC, ROWS_PAD, D), jnp.float32),
        mesh=mesh,
        scratch_types=[
            pltpu.VMEM((HALF, CHUNK), jnp.int32),
            pltpu.VMEM((HALF, CHUNK), jnp.int32),
            pltpu.VMEM((CHUNK, D), jnp.float32),
            pltpu.VMEM((CHUNK, D), jnp.float32),
            pltpu.VMEM_SHARED((ROWS_PAD, D), jnp.float32),
            pltpu.SemaphoreType.DMA,
            pltpu.SemaphoreType.DMA,
        ],
    )
    def edge_scatter(y_hbm, src_hbm, dst_hbm, zeros_hbm, out_hbm,
                     src_v, dst_v, rows_a, rows_b, acc_sh, gsem, ssem):
        c = lax.axis_index("c")
        s = lax.axis_index("s")
        stripe = pl.ds(s * ROWS_PER_TILE, ROWS_PER_TILE)
        pltpu.sync_copy(zeros_hbm, acc_sh.at[stripe])

        def gather(j, buf):
            pltpu.async_copy(y_hbm.at[src_v.at[j]], buf, gsem)

        def gather_wait(j, buf):
            pltpu.make_async_copy(y_hbm.at[src_v.at[j]], buf, gsem).wait()

        def scatter(j, buf):
            pltpu.async_copy(buf, acc_sh.at[dst_v.at[j]], ssem, add=True)

        def scatter_wait(j, buf):
            pltpu.make_async_copy(buf, acc_sh.at[dst_v.at[j]], ssem).wait()

        for p in range(2):
            pltpu.sync_copy(src_hbm.at[c, s, pl.ds(p * HALF, HALF)], src_v)
            pltpu.sync_copy(dst_hbm.at[c, s, pl.ds(p * HALF, HALF)], dst_v)
            if p == 0:
                plsc.subcore_barrier()

            gather(0, rows_a)
            gather_wait(0, rows_a)
            scatter(0, rows_a)
            gather(1, rows_b)
            gather_wait(1, rows_b)
            scatter(1, rows_b)
            scatter_wait(0, rows_a)
            gather(2, rows_a)

            @pl.loop(2, HALF - 2, step=2)
            def _(j):
                gather_wait(j, rows_a)
                scatter(j, rows_a)
                scatter_wait(j - 1, rows_b)
                gather(j + 1, rows_b)
                gather_wait(j + 1, rows_b)
                scatter(j + 1, rows_b)
                scatter_wait(j, rows_a)
                gather(j + 2, rows_a)

            gather_wait(HALF - 2, rows_a)
            scatter(HALF - 2, rows_a)
            scatter_wait(HALF - 3, rows_b)
            gather(HALF - 1, rows_b)
            gather_wait(HALF - 1, rows_b)
            scatter(HALF - 1, rows_b)
            scatter_wait(HALF - 2, rows_a)
            scatter_wait(HALF - 1, rows_b)

        plsc.subcore_barrier()
        pltpu.sync_copy(acc_sh.at[stripe], out_hbm.at[c, stripe])

    return edge_scatter


_edge_scatter = _make_edge_scatter()


def kernel(x, edge_index, W1, b1, W2, b2):
    src = edge_index[0].astype(jnp.int32)
    dst = edge_index[1].astype(jnp.int32)
    padn = E_PAD - E
    pad_src = jnp.arange(padn, dtype=jnp.int32) % N
    src_p = jnp.concatenate([src, pad_src]).reshape(NC, NS, CH_PER_TILE, CHUNK)
    dummy = N + (jnp.arange(padn, dtype=jnp.int32) % (ROWS_PAD - N))
    dst_p = jnp.concatenate([dst, dummy]).reshape(NC, NS, CH_PER_TILE, CHUNK)
    zeros_tile = jnp.zeros((ROWS_PER_TILE, D), jnp.float32)
    b1_t = jnp.broadcast_to(b1.reshape(1, D), (8, D))
    b2_t = jnp.broadcast_to(b2.reshape(1, D), (8, D))

    y1 = _matmul(x, W1)
    p1 = _edge_scatter(y1, src_p, dst_p, zeros_tile)
    y2 = _fused_relu_mm(p1, b1_t, W2)
    p2 = _edge_scatter(y2, src_p, dst_p, zeros_tile)
    return _final_add(p2, b2_t)
